# baseline (speedup 1.0000x reference)
import functools

import jax
import jax.numpy as jnp
from jax import lax
from jax.experimental import pallas as pl
from jax.experimental.pallas import tpu as pltpu
from jax.experimental.pallas import tpu_sc as plsc

D = 1024
L = 16
NW = 32
ROWS = 32768
ROWS_PER_W = ROWS // NW
K = 16
NCHUNK = ROWS_PER_W // K
NB = 4
NPB = 2
S_LEN = 8192

_mesh = plsc.VectorSubcoreMesh(core_axis_name="c", subcore_axis_name="s")


@functools.partial(
    pl.kernel,
    mesh=_mesh,
    out_type=jax.ShapeDtypeStruct((ROWS, D), jnp.float32),
    scratch_types=[
        pltpu.VMEM((NCHUNK, K), jnp.int32),
        pltpu.VMEM((NB, K, D), jnp.float32),
        pltpu.VMEM((NPB, K, D), jnp.float32),
        pltpu.SemaphoreType.DMA((NB,)),
        pltpu.SemaphoreType.DMA((NPB,)),
        pltpu.SemaphoreType.DMA((NB,)),
    ],
)
def _emb_kernel(idx_hbm, tok_hbm, pos_hbm, out_hbm,
                idx_v, rows_v, pos_v, sem_g, sem_p, sem_o):
    cid = lax.axis_index("c")
    sid = lax.axis_index("s")
    wid = sid * 2 + cid
    base = wid * ROWS_PER_W
    s0 = lax.rem(base, S_LEN)

    pltpu.sync_copy(idx_hbm.at[pl.ds(wid * NCHUNK, NCHUNK)], idx_v)

    def gather(ci, slot):
        pltpu.async_copy(tok_hbm.at[idx_v.at[ci]], rows_v.at[slot],
                         sem_g.at[slot])

    def posload(ci, slot):
        pltpu.async_copy(pos_hbm.at[pl.ds(s0 + ci * K, K)], pos_v.at[slot],
                         sem_p.at[slot])

    def outwrite(ci, slot):
        pltpu.async_copy(rows_v.at[slot], out_hbm.at[pl.ds(base + ci * K, K)],
                         sem_o.at[slot])

    def wait_out(ci, slot):
        pltpu.make_async_copy(rows_v.at[slot],
                              out_hbm.at[pl.ds(base + ci * K, K)],
                              sem_o.at[slot]).wait()

    gather(0, 0)
    gather(1, 1)
    posload(0, 0)

    def body(ci, carry):
        b = lax.rem(ci, NB)
        bp = lax.rem(ci, NPB)
        bn = lax.rem(ci + 2, NB)
        bpn = lax.rem(ci + 1, NPB)

        @pl.when(ci >= 2)
        def _():
            wait_out(ci - 2, bn)

        @pl.when(ci < NCHUNK - 2)
        def _():
            gather(ci + 2, bn)

        @pl.when(ci < NCHUNK - 1)
        def _():
            posload(ci + 1, bpn)

        pltpu.make_async_copy(tok_hbm.at[idx_v.at[ci]], rows_v.at[b],
                              sem_g.at[b]).wait()
        pltpu.make_async_copy(pos_hbm.at[pl.ds(s0 + ci * K, K)],
                              pos_v.at[bp], sem_p.at[bp]).wait()

        def row_body(i, c2):
            for j in range(D // L):
                sl = pl.ds(j * L, L)
                rows_v[b, i, sl] = rows_v[b, i, sl] + pos_v[bp, i, sl]
            return c2

        lax.fori_loop(0, K, row_body, 0)
        outwrite(ci, b)
        return carry

    lax.fori_loop(0, NCHUNK, body, 0)

    wait_out(NCHUNK - 2, (NCHUNK - 2) % NB)
    wait_out(NCHUNK - 1, (NCHUNK - 1) % NB)


def kernel(x, token_table, pos_table):
    b, s = x.shape
    idx = x.reshape(ROWS).astype(jnp.int32).reshape(NW * NCHUNK, K)
    out = _emb_kernel(idx, token_table, pos_table)
    return out.reshape(b, s, D)


# speedup vs baseline: 1.2955x; 1.2955x over previous
import functools

import jax
import jax.numpy as jnp
from jax import lax
from jax.experimental import pallas as pl
from jax.experimental.pallas import tpu as pltpu
from jax.experimental.pallas import tpu_sc as plsc

D = 1024
L = 16
NW = 32
B = 4
ROWS = 32768
S_LEN = 8192
S_PER_W = S_LEN // NW
K = 16
NSC = S_PER_W // K
NCHUNK = NSC * B
NB = 4
NPB = 2

_mesh = plsc.VectorSubcoreMesh(core_axis_name="c", subcore_axis_name="s")


@functools.partial(
    pl.kernel,
    mesh=_mesh,
    out_type=jax.ShapeDtypeStruct((ROWS, D), jnp.float32),
    scratch_types=[
        pltpu.VMEM((NCHUNK, K), jnp.int32),
        pltpu.VMEM((NB, K, D), jnp.float32),
        pltpu.VMEM((NPB, K, D), jnp.float32),
        pltpu.SemaphoreType.DMA((NB,)),
        pltpu.SemaphoreType.DMA((NPB,)),
        pltpu.SemaphoreType.DMA((NB,)),
    ],
)
def _emb_kernel(idx_hbm, tok_hbm, pos_hbm, out_hbm,
                idx_v, rows_v, pos_v, sem_g, sem_p, sem_o):
    cid = lax.axis_index("c")
    sid = lax.axis_index("s")
    wid = sid * 2 + cid
    w_s0 = wid * S_PER_W

    pltpu.sync_copy(idx_hbm.at[pl.ds(wid * NCHUNK, NCHUNK)], idx_v)

    def out_off(ci):
        sc = lax.div(ci, B)
        b = lax.rem(ci, B)
        return b * S_LEN + w_s0 + sc * K

    def gather(ci, slot):
        pltpu.async_copy(tok_hbm.at[idx_v.at[ci]], rows_v.at[slot],
                         sem_g.at[slot])

    def posload(sc, slot):
        pltpu.async_copy(pos_hbm.at[pl.ds(w_s0 + sc * K, K)], pos_v.at[slot],
                         sem_p.at[slot])

    def outwrite(ci, slot):
        pltpu.async_copy(rows_v.at[slot], out_hbm.at[pl.ds(out_off(ci), K)],
                         sem_o.at[slot])

    def wait_out(ci, slot):
        pltpu.make_async_copy(rows_v.at[slot],
                              out_hbm.at[pl.ds(out_off(ci), K)],
                              sem_o.at[slot]).wait()

    gather(0, 0)
    gather(1, 1)
    posload(0, 0)

    def body(ci, carry):
        sc = lax.div(ci, B)
        b = lax.rem(ci, B)
        slot = lax.rem(ci, NB)
        sp = lax.rem(sc, NPB)
        sn = lax.rem(ci + 2, NB)

        @pl.when(ci >= 2)
        def _():
            wait_out(ci - 2, sn)

        @pl.when(ci < NCHUNK - 2)
        def _():
            gather(ci + 2, sn)

        @pl.when((b == 0) & (sc < NSC - 1))
        def _():
            posload(sc + 1, lax.rem(sc + 1, NPB))

        pltpu.make_async_copy(tok_hbm.at[idx_v.at[ci]], rows_v.at[slot],
                              sem_g.at[slot]).wait()

        @pl.when(b == 0)
        def _():
            pltpu.make_async_copy(pos_hbm.at[pl.ds(w_s0 + sc * K, K)],
                                  pos_v.at[sp], sem_p.at[sp]).wait()

        def row_body(i, c2):
            for j in range(D // L):
                sl = pl.ds(j * L, L)
                plsc.addupdate(rows_v.at[slot, i, sl], pos_v[sp, i, sl])
            return c2

        lax.fori_loop(0, K, row_body, 0)
        outwrite(ci, slot)
        return carry

    lax.fori_loop(0, NCHUNK, body, 0)

    wait_out(NCHUNK - 2, (NCHUNK - 2) % NB)
    wait_out(NCHUNK - 1, (NCHUNK - 1) % NB)


def kernel(x, token_table, pos_table):
    b, s = x.shape
    idx = (x.astype(jnp.int32)
           .reshape(B, NW, NSC, K)
           .transpose(1, 2, 0, 3)
           .reshape(NW * NCHUNK, K))
    out = _emb_kernel(idx, token_table, pos_table)
    return out.reshape(b, s, D)


# speedup vs baseline: 2.5134x; 1.9402x over previous
import functools

import jax
import jax.numpy as jnp
from jax import lax
from jax.experimental import pallas as pl
from jax.experimental.pallas import tpu as pltpu
from jax.experimental.pallas import tpu_sc as plsc

D = 1024
L = 16
NW = 32
B = 4
ROWS = 32768
S_LEN = 8192
S_PER_W = S_LEN // NW
K = 8
NSC = S_PER_W // K
NG = 3
NPB = 2
NQ = 4
QV = D // L // NQ

_mesh = plsc.VectorSubcoreMesh(core_axis_name="c", subcore_axis_name="s")


@functools.partial(
    pl.kernel,
    mesh=_mesh,
    out_type=jax.ShapeDtypeStruct((ROWS, D), jnp.float32),
    scratch_types=[
        pltpu.VMEM((NSC, B, K), jnp.int32),
        pltpu.VMEM((NG, B, K, D), jnp.float32),
        pltpu.VMEM((NPB, K, D), jnp.float32),
        pltpu.SemaphoreType.DMA((NG * B,)),
        pltpu.SemaphoreType.DMA((NPB,)),
        pltpu.SemaphoreType.DMA((NG * B,)),
    ],
)
def _emb_kernel(idx_hbm, tok_hbm, pos_hbm, out_hbm,
                idx_v, rows_v, pos_v, sem_g, sem_p, sem_o):
    cid = lax.axis_index("c")
    sid = lax.axis_index("s")
    wid = sid * 2 + cid
    w_s0 = wid * S_PER_W

    pltpu.sync_copy(idx_hbm.at[pl.ds(wid * NSC, NSC)], idx_v)

    def gather(sc, b, g):
        pltpu.async_copy(tok_hbm.at[idx_v.at[sc, b]], rows_v.at[g, b],
                         sem_g.at[g * B + b])

    def wait_gather(sc, b, g):
        pltpu.make_async_copy(tok_hbm.at[idx_v.at[sc, b]], rows_v.at[g, b],
                              sem_g.at[g * B + b]).wait()

    def posload(sc, slot):
        pltpu.async_copy(pos_hbm.at[pl.ds(w_s0 + sc * K, K)], pos_v.at[slot],
                         sem_p.at[slot])

    def wait_pos(sc, slot):
        pltpu.make_async_copy(pos_hbm.at[pl.ds(w_s0 + sc * K, K)],
                              pos_v.at[slot], sem_p.at[slot]).wait()

    def outwrite(sc, b, g):
        pltpu.async_copy(rows_v.at[g, b],
                         out_hbm.at[pl.ds(b * S_LEN + w_s0 + sc * K, K)],
                         sem_o.at[g * B + b])

    def wait_out(sc, b, g):
        pltpu.make_async_copy(rows_v.at[g, b],
                              out_hbm.at[pl.ds(b * S_LEN + w_s0 + sc * K, K)],
                              sem_o.at[g * B + b]).wait()

    for b in range(B):
        gather(0, b, 0)
    for b in range(B):
        gather(1, b, 1)
    posload(0, 0)

    def body(sc, carry):
        g = lax.rem(sc, NG)
        gp = lax.rem(sc, NPB)
        gt = lax.rem(sc + 2, NG)

        @pl.when((sc >= 1) & (sc < NSC - 2))
        def _():
            for b in range(B):
                wait_out(sc - 1, b, gt)
            for b in range(B):
                gather(sc + 2, b, gt)

        @pl.when(sc == 0)
        def _():
            for b in range(B):
                gather(2, b, gt)

        @pl.when(sc < NSC - 1)
        def _():
            posload(sc + 1, lax.rem(sc + 1, NPB))

        for b in range(B):
            wait_gather(sc, b, g)
        wait_pos(sc, gp)

        def row_body(i, c2):
            for q in range(NQ):
                pv = [pos_v[gp, i, pl.ds((q * QV + j) * L, L)]
                      for j in range(QV)]
                for b in range(B):
                    for j in range(QV):
                        plsc.addupdate(
                            rows_v.at[g, b, i, pl.ds((q * QV + j) * L, L)],
                            pv[j])
            return c2

        lax.fori_loop(0, K, row_body, 0)

        for b in range(B):
            outwrite(sc, b, g)
        return carry

    lax.fori_loop(0, NSC, body, 0)

    for sc in (NSC - 3, NSC - 2, NSC - 1):
        for b in range(B):
            wait_out(sc, b, sc % NG)


def kernel(x, token_table, pos_table):
    b, s = x.shape
    idx = (x.astype(jnp.int32)
           .reshape(B, NW, NSC, K)
           .transpose(1, 2, 0, 3)
           .reshape(NW * NSC, B, K))
    out = _emb_kernel(idx, token_table, pos_table)
    return out.reshape(b, s, D)
